# baseline (speedup 1.0000x reference)
import functools

import jax
import jax.numpy as jnp
from jax import lax
from jax.experimental import pallas as pl
from jax.experimental.pallas import tpu as pltpu
from jax.experimental.pallas import tpu_sc as plsc

N = 10000
D = 128
ED = 16
G = 64
NC = 2
NS = 16
NW = NC * NS
CH = 64
NPAD = 10112
RPT = NPAD // NS

def _mesh():
    return plsc.VectorSubcoreMesh(core_axis_name="c", subcore_axis_name="s",
                                  num_cores=NC, num_subcores=NS)


_NB = 4
_NP = 2


def _sc_scatter_h_body(cpw, h_hbm, src_hbm, dst_hbm, zeros_hbm, out_hbm,
                       idx_s, idx_d, rows, acc, *sems):
    c = lax.axis_index("c")
    s = lax.axis_index("s")
    hp = cpw // _NP
    pltpu.sync_copy(zeros_hbm.at[pl.ds(s * RPT, RPT)],
                    acc.at[pl.ds(s * RPT, RPT)])
    plsc.subcore_barrier()
    base = (c * NS + s) * cpw

    gsem = sems[:_NB]
    ssem = sems[_NB:]
    for p in range(_NP):
        pltpu.sync_copy(src_hbm.at[pl.ds(base + p * hp, hp)], idx_s)
        pltpu.sync_copy(dst_hbm.at[pl.ds(base + p * hp, hp)], idx_d)
        for b in range(_NB - 1):
            pltpu.async_copy(h_hbm.at[idx_s.at[b]], rows.at[b], gsem[b])

        @pl.loop(0, hp // _NB)
        def _(o):
            j0 = o * _NB
            for b in range(_NB):
                j = j0 + b
                nxt = j + _NB - 1
                nb = (_NB - 1 + b) % _NB

                @pl.when(nxt < hp)
                def _():
                    @pl.when(j >= 1)
                    def _():
                        pltpu.make_async_copy(
                            rows.at[nb], acc.at[idx_d.at[j]],
                            ssem[nb]).wait()

                    pltpu.async_copy(h_hbm.at[idx_s.at[nxt]], rows.at[nb],
                                     gsem[nb])

                pltpu.make_async_copy(h_hbm.at[idx_s.at[j]], rows.at[b],
                                      gsem[b]).wait()
                pltpu.async_copy(rows.at[b], acc.at[idx_d.at[j]], ssem[b],
                                 add=True)

        for b in range(_NB):
            pltpu.make_async_copy(rows.at[b], acc.at[idx_d.at[b]],
                                  ssem[b]).wait()

    plsc.subcore_barrier()
    pltpu.sync_copy(acc.at[pl.ds(s * RPT, RPT)],
                    out_hbm.at[c, pl.ds(s * RPT, RPT)])


def _make_scatter_h(cpw):
    return pl.kernel(
        functools.partial(_sc_scatter_h_body, cpw),
        out_type=jax.ShapeDtypeStruct((NC, NPAD, D), jnp.float32),
        mesh=_mesh(),
        scratch_types=[
            pltpu.VMEM((cpw // _NP, CH), jnp.int32),
            pltpu.VMEM((cpw // _NP, CH), jnp.int32),
            pltpu.VMEM((_NB, CH, D), jnp.float32),
            pltpu.VMEM_SHARED((NPAD, D), jnp.float32),
        ] + [pltpu.SemaphoreType.DMA] * (2 * _NB),
    )


def _bn_relu(y, g, b):
    mu = jnp.mean(y, axis=0, keepdims=True)
    var = jnp.mean((y - mu) * (y - mu), axis=0, keepdims=True)
    return jnp.maximum(g * (y - mu) * lax.rsqrt(var + 1e-5) + b, 0.0)


def _tc_prep_body(x_ref, win_ref, bin_ref, auxp_ref, h_ref, aux_ref):
    h_ref[...] = (jnp.dot(x_ref[...], win_ref[...],
                          preferred_element_type=jnp.float32) + bin_ref[...])
    aux_ref[...] = auxp_ref[0, :N, :32] + auxp_ref[1, :N, :32]


def _tc_layer_body(h_ref, sp_ref, aux_ref, w1_ref, b1_ref, g1_ref, be1_ref,
                   w2_ref, b2_ref, g2_ref, be2_ref, out_ref):
    h = h_ref[...]
    s_agg = sp_ref[0, :N, :] + sp_ref[1, :N, :]
    aux = aux_ref[...]
    ea = aux[:, :ED]
    deg = aux[:, ED:ED + 1]
    w1 = w1_ref[...]
    y = jnp.dot(h, w1[:D, :], preferred_element_type=jnp.float32)
    y += jnp.dot(h * deg, w1[D:2 * D, :], preferred_element_type=jnp.float32)
    y += jnp.dot(s_agg, w1[2 * D:3 * D, :], preferred_element_type=jnp.float32)
    y += jnp.dot(ea, w1[3 * D:, :], preferred_element_type=jnp.float32)
    y += b1_ref[...]
    y = _bn_relu(y, g1_ref[...], be1_ref[...])
    y = jnp.dot(y, w2_ref[...], preferred_element_type=jnp.float32) + b2_ref[...]
    y = _bn_relu(y, g2_ref[...], be2_ref[...])
    out_ref[...] = h + y


def _tc_pool_body(h_ref, batch_ref, wp_ref, bp_ref, out_ref):
    ids = lax.broadcasted_iota(jnp.int32, (G, N), 0)
    oht = (ids == batch_ref[...]).astype(jnp.float32)
    sums = jnp.dot(oht, h_ref[...], preferred_element_type=jnp.float32)
    counts = jnp.sum(oht, axis=1, keepdims=True)
    hg = sums / jnp.maximum(counts, 1.0)
    out_ref[...] = jnp.dot(hg, wp_ref[...],
                           preferred_element_type=jnp.float32) + bp_ref[...]


_tc_params = pltpu.CompilerParams(vmem_limit_bytes=128 * 1024 * 1024)

_prep_call = pl.pallas_call(
    _tc_prep_body,
    out_shape=(jax.ShapeDtypeStruct((N, D), jnp.float32),
               jax.ShapeDtypeStruct((N, 32), jnp.float32)),
    compiler_params=_tc_params,
)

_layer_call = pl.pallas_call(
    _tc_layer_body,
    out_shape=jax.ShapeDtypeStruct((N, D), jnp.float32),
    compiler_params=_tc_params,
)

_pool_call = pl.pallas_call(
    _tc_pool_body,
    out_shape=jax.ShapeDtypeStruct((G, 1), jnp.float32),
    compiler_params=_tc_params,
)


def kernel(x, edge_index, edge_attr, batch, Win, b_in, W1, b1, gamma1, beta1,
           W2, b2, gamma2, beta2, Wpred, bpred):
    e = edge_index.shape[1]
    q = NW * CH * _NB * _NP
    epad = -(-e // q) * q
    cpw = epad // (NW * CH)
    nrows = epad // CH
    src = edge_index[0]
    dst = edge_index[1]
    pad = epad - e
    src_p = jnp.concatenate(
        [src, jnp.zeros((pad,), jnp.int32)]).reshape(nrows, CH)
    dst_p = jnp.concatenate(
        [dst, jnp.full((pad,), N, jnp.int32)]).reshape(nrows, CH)
    ea128 = jnp.concatenate(
        [edge_attr, jnp.ones((e, 1), jnp.float32),
         jnp.zeros((e, D - ED - 1), jnp.float32)], axis=1)
    ea128 = jnp.concatenate(
        [ea128, jnp.zeros((pad, D), jnp.float32)], axis=0)
    iota_p = jnp.arange(epad, dtype=jnp.int32).reshape(nrows, CH)
    zeros_big = jnp.zeros((NPAD, D), jnp.float32)

    scatter_h = _make_scatter_h(cpw)
    aux_part = scatter_h(ea128, iota_p, dst_p, zeros_big)
    h, aux = _prep_call(x, Win, b_in, aux_part)
    for l in range(W1.shape[0]):
        sp = scatter_h(h, src_p, dst_p, zeros_big)
        h = _layer_call(h, sp, aux, W1[l], b1[l], gamma1[l], beta1[l],
                        W2[l], b2[l], gamma2[l], beta2[l])
    out = _pool_call(h, batch.reshape(1, N), Wpred, bpred)
    return out.reshape(-1)


# speedup vs baseline: 1.1436x; 1.1436x over previous
import functools

import jax
import jax.numpy as jnp
from jax import lax
from jax.experimental import pallas as pl
from jax.experimental.pallas import tpu as pltpu
from jax.experimental.pallas import tpu_sc as plsc

N = 10000
D = 128
ED = 16
G = 64
NC = 2
NS = 16
NW = NC * NS
CH = 32
NPAD = 10112
RPT = NPAD // NS

def _mesh():
    return plsc.VectorSubcoreMesh(core_axis_name="c", subcore_axis_name="s",
                                  num_cores=NC, num_subcores=NS)


_NB = 5
_NP = 2


def _sc_scatter_h_body(cpw, h_hbm, src_hbm, dst_hbm, zeros_hbm, out_hbm,
                       idx_s, idx_d, rows, acc, *sems):
    c = lax.axis_index("c")
    s = lax.axis_index("s")
    hp = cpw // _NP
    pltpu.sync_copy(zeros_hbm.at[pl.ds(s * RPT, RPT)],
                    acc.at[pl.ds(s * RPT, RPT)])
    plsc.subcore_barrier()
    base = (c * NS + s) * cpw

    gsem = sems[:_NB]
    ssem = sems[_NB:]
    for p in range(_NP):
        pltpu.sync_copy(src_hbm.at[pl.ds(base + p * hp, hp)], idx_s)
        pltpu.sync_copy(dst_hbm.at[pl.ds(base + p * hp, hp)], idx_d)
        for b in range(_NB - 1):
            pltpu.async_copy(h_hbm.at[idx_s.at[b]], rows.at[b], gsem[b])

        @pl.loop(0, hp // _NB)
        def _(o):
            j0 = o * _NB
            for b in range(_NB):
                j = j0 + b
                nxt = j + _NB - 1
                nb = (_NB - 1 + b) % _NB

                @pl.when(nxt < hp)
                def _():
                    @pl.when(j >= 1)
                    def _():
                        pltpu.make_async_copy(
                            rows.at[nb], acc.at[idx_d.at[j]],
                            ssem[nb]).wait()

                    pltpu.async_copy(h_hbm.at[idx_s.at[nxt]], rows.at[nb],
                                     gsem[nb])

                pltpu.make_async_copy(h_hbm.at[idx_s.at[j]], rows.at[b],
                                      gsem[b]).wait()
                pltpu.async_copy(rows.at[b], acc.at[idx_d.at[j]], ssem[b],
                                 add=True)

        for b in range(_NB):
            pltpu.make_async_copy(rows.at[b], acc.at[idx_d.at[b]],
                                  ssem[b]).wait()

    plsc.subcore_barrier()
    pltpu.sync_copy(acc.at[pl.ds(s * RPT, RPT)],
                    out_hbm.at[c, pl.ds(s * RPT, RPT)])


def _make_scatter_h(cpw):
    return pl.kernel(
        functools.partial(_sc_scatter_h_body, cpw),
        out_type=jax.ShapeDtypeStruct((NC, NPAD, D), jnp.float32),
        mesh=_mesh(),
        scratch_types=[
            pltpu.VMEM((cpw // _NP, CH), jnp.int32),
            pltpu.VMEM((cpw // _NP, CH), jnp.int32),
            pltpu.VMEM((_NB, CH, D), jnp.float32),
            pltpu.VMEM_SHARED((NPAD, D), jnp.float32),
        ] + [pltpu.SemaphoreType.DMA] * (2 * _NB),
    )


def _bn_relu(y, g, b):
    mu = jnp.mean(y, axis=0, keepdims=True)
    var = jnp.mean((y - mu) * (y - mu), axis=0, keepdims=True)
    return jnp.maximum(g * (y - mu) * lax.rsqrt(var + 1e-5) + b, 0.0)


def _tc_prep_body(x_ref, win_ref, bin_ref, auxp_ref, h_ref, aux_ref):
    h_ref[...] = (jnp.dot(x_ref[...], win_ref[...],
                          preferred_element_type=jnp.float32) + bin_ref[...])
    aux_ref[...] = auxp_ref[0, :N, :32] + auxp_ref[1, :N, :32]


def _tc_layer_body(h_ref, sp_ref, aux_ref, w1_ref, b1_ref, g1_ref, be1_ref,
                   w2_ref, b2_ref, g2_ref, be2_ref, out_ref):
    h = h_ref[...]
    s_agg = sp_ref[0, :N, :] + sp_ref[1, :N, :]
    aux = aux_ref[...]
    ea = aux[:, :ED]
    deg = aux[:, ED:ED + 1]
    w1 = w1_ref[...]
    y = jnp.dot(h, w1[:D, :], preferred_element_type=jnp.float32)
    y += jnp.dot(h * deg, w1[D:2 * D, :], preferred_element_type=jnp.float32)
    y += jnp.dot(s_agg, w1[2 * D:3 * D, :], preferred_element_type=jnp.float32)
    y += jnp.dot(ea, w1[3 * D:, :], preferred_element_type=jnp.float32)
    y += b1_ref[...]
    y = _bn_relu(y, g1_ref[...], be1_ref[...])
    y = jnp.dot(y, w2_ref[...], preferred_element_type=jnp.float32) + b2_ref[...]
    y = _bn_relu(y, g2_ref[...], be2_ref[...])
    out_ref[...] = h + y


def _tc_pool_body(h_ref, batch_ref, wp_ref, bp_ref, out_ref):
    ids = lax.broadcasted_iota(jnp.int32, (G, N), 0)
    oht = (ids == batch_ref[...]).astype(jnp.float32)
    sums = jnp.dot(oht, h_ref[...], preferred_element_type=jnp.float32)
    counts = jnp.sum(oht, axis=1, keepdims=True)
    hg = sums / jnp.maximum(counts, 1.0)
    out_ref[...] = jnp.dot(hg, wp_ref[...],
                           preferred_element_type=jnp.float32) + bp_ref[...]


_tc_params = pltpu.CompilerParams(vmem_limit_bytes=128 * 1024 * 1024)

_prep_call = pl.pallas_call(
    _tc_prep_body,
    out_shape=(jax.ShapeDtypeStruct((N, D), jnp.float32),
               jax.ShapeDtypeStruct((N, 32), jnp.float32)),
    compiler_params=_tc_params,
)

_layer_call = pl.pallas_call(
    _tc_layer_body,
    out_shape=jax.ShapeDtypeStruct((N, D), jnp.float32),
    compiler_params=_tc_params,
)

_pool_call = pl.pallas_call(
    _tc_pool_body,
    out_shape=jax.ShapeDtypeStruct((G, 1), jnp.float32),
    compiler_params=_tc_params,
)


def kernel(x, edge_index, edge_attr, batch, Win, b_in, W1, b1, gamma1, beta1,
           W2, b2, gamma2, beta2, Wpred, bpred):
    e = edge_index.shape[1]
    q = NW * CH * _NB * _NP
    epad = -(-e // q) * q
    cpw = epad // (NW * CH)
    nrows = epad // CH
    src = edge_index[0]
    dst = edge_index[1]
    pad = epad - e
    src_p = jnp.concatenate(
        [src, jnp.zeros((pad,), jnp.int32)]).reshape(nrows, CH)
    dst_p = jnp.concatenate(
        [dst, jnp.full((pad,), N, jnp.int32)]).reshape(nrows, CH)
    ea128 = jnp.concatenate(
        [edge_attr, jnp.ones((e, 1), jnp.float32),
         jnp.zeros((e, D - ED - 1), jnp.float32)], axis=1)
    ea128 = jnp.concatenate(
        [ea128, jnp.zeros((pad, D), jnp.float32)], axis=0)
    iota_p = jnp.arange(epad, dtype=jnp.int32).reshape(nrows, CH)
    zeros_big = jnp.zeros((NPAD, D), jnp.float32)

    scatter_h = _make_scatter_h(cpw)
    aux_part = scatter_h(ea128, iota_p, dst_p, zeros_big)
    h, aux = _prep_call(x, Win, b_in, aux_part)
    for l in range(W1.shape[0]):
        sp = scatter_h(h, src_p, dst_p, zeros_big)
        h = _layer_call(h, sp, aux, W1[l], b1[l], gamma1[l], beta1[l],
                        W2[l], b2[l], gamma2[l], beta2[l])
    out = _pool_call(h, batch.reshape(1, N), Wpred, bpred)
    return out.reshape(-1)
